# baseline (speedup 1.0000x reference)
import jax
import jax.numpy as jnp
from jax.experimental import pallas as pl

_N = 8192
_D = 1000
_K = 8
_INV_TEMP = 0.5
_BLOCK_ROWS = 512


_DP = 1024
_NG = _DP // 128


def _topk_mask_kernel(cls_ref, out_ref):
    x = cls_ref[:]
    m = jnp.max(x, axis=-1, keepdims=True)
    ew = jnp.exp((x - m) * _INV_TEMP)
    s = jnp.sum(ew, axis=-1, keepdims=True)
    rows = ew.shape[0]
    ewp = jnp.concatenate(
        [ew, jnp.zeros((rows, _DP - _D), jnp.float32)], axis=1
    )
    ckey = (
        _DP - jax.lax.broadcasted_iota(jnp.int32, ewp.shape, 1)
    ).astype(jnp.float32)

    def rebuild(ewp):
        g = ewp[:, 0:128]
        c = ckey[:, 0:128]
        for k in range(1, _NG):
            e_k = ewp[:, k * 128:(k + 1) * 128]
            take = e_k > g
            g = jnp.where(take, e_k, g)
            c = jnp.where(take, ckey[:, k * 128:(k + 1) * 128], c)
        return g, c

    gmax, gkey = rebuild(ewp)
    for i in range(_K):
        v = jnp.max(gmax, axis=-1, keepdims=True)
        wk = jnp.max(jnp.where(gmax >= v, gkey, 0.0), axis=-1, keepdims=True)
        ewp = jnp.where(ckey == wk, -v / s, ewp)
        s = s - v
        if i < _K - 1:
            gmax, gkey = rebuild(ewp)
    out_ref[:] = -jnp.minimum(ewp[:, :_D], 0.0)


def kernel(normu, cls):
    classes = pl.pallas_call(
        _topk_mask_kernel,
        grid=(_N // _BLOCK_ROWS,),
        in_specs=[pl.BlockSpec((_BLOCK_ROWS, _D), lambda i: (i, 0))],
        out_specs=pl.BlockSpec((_BLOCK_ROWS, _D), lambda i: (i, 0)),
        out_shape=jax.ShapeDtypeStruct((_N, _D), jnp.float32),
    )(cls)
    return (normu, classes)
